# baseline (speedup 1.0000x reference)
import functools

import jax
import jax.numpy as jnp
from jax import lax
from jax.experimental import pallas as pl
from jax.experimental.pallas import tpu as pltpu
from jax.experimental.pallas import tpu_sc as plsc

BATCH = 4096
SEQ = 200
N = BATCH * SEQ
D_ITEM = 64
D_FAKE = 16
D_OUT = D_ITEM + D_FAKE
NUM_WORKERS = 32
B_PER_W = BATCH // NUM_WORKERS
SPLIT_A = 104
SPLIT_B = SEQ - SPLIT_A


def _sc_embed(ids, fids, item_table, pos_table, fake_table, fake_dummy):
    mesh = plsc.VectorSubcoreMesh(core_axis_name="c", subcore_axis_name="s")

    @functools.partial(
        pl.kernel,
        mesh=mesh,
        compiler_params=pltpu.CompilerParams(use_tc_tiling_on_sc=False),
        out_type=jax.ShapeDtypeStruct((N, D_OUT), jnp.float32),
        scratch_types=[
            pltpu.VMEM((B_PER_W, SEQ), jnp.int32),
            pltpu.VMEM((B_PER_W, SEQ), jnp.int32),
            pltpu.VMEM((SEQ, D_ITEM), jnp.float32),
            pltpu.VMEM((SEQ, D_ITEM), jnp.float32),
            pltpu.VMEM((SEQ, D_ITEM), jnp.float32),
            pltpu.VMEM((SEQ, D_FAKE), jnp.float32),
            pltpu.VMEM((SEQ, D_FAKE), jnp.float32),
            pltpu.SemaphoreType.DMA,
            pltpu.SemaphoreType.DMA,
            pltpu.SemaphoreType.DMA,
            pltpu.SemaphoreType.DMA,
        ],
    )
    def k(ids_hbm, fids_hbm, item_hbm, pos_hbm, fake_hbm, fdum_hbm, out_hbm,
          ids_v, fids_v, pos_v, it0_v, it1_v, fk0_v, fk1_v, gs0, gs1, ws0, ws1):
        wid = lax.axis_index("s") * 2 + lax.axis_index("c")
        b0 = wid * B_PER_W
        pltpu.sync_copy(pos_hbm, pos_v)
        pltpu.sync_copy(ids_hbm.at[pl.ds(b0, B_PER_W)], ids_v)
        pltpu.sync_copy(fids_hbm.at[pl.ds(b0, B_PER_W)], fids_v)

        def issue_gathers(c, ibuf, fbuf, sem):
            return (
                pltpu.async_copy(
                    item_hbm.at[ids_v.at[c, pl.ds(0, SPLIT_A)]],
                    ibuf.at[pl.ds(0, SPLIT_A)], sem),
                pltpu.async_copy(
                    item_hbm.at[ids_v.at[c, pl.ds(SPLIT_A, SPLIT_B)]],
                    ibuf.at[pl.ds(SPLIT_A, SPLIT_B)], sem),
                pltpu.async_copy(
                    fake_hbm.at[fids_v.at[c, pl.ds(0, SPLIT_A)]],
                    fbuf.at[pl.ds(0, SPLIT_A)], sem),
                pltpu.async_copy(
                    fake_hbm.at[fids_v.at[c, pl.ds(SPLIT_A, SPLIT_B)]],
                    fbuf.at[pl.ds(SPLIT_A, SPLIT_B)], sem),
            )

        def issue_writeback(c, ibuf, fbuf, sem):
            rows = pl.ds((b0 + c) * SEQ, SEQ)
            return (
                pltpu.async_copy(ibuf, out_hbm.at[rows, pl.ds(0, D_ITEM)], sem),
                pltpu.async_copy(fbuf, out_hbm.at[rows, pl.ds(D_ITEM, D_FAKE)], sem),
            )

        def pos_add(ibuf):
            def row_body(s, carry):
                for j in range(D_ITEM // 16):
                    plsc.addupdate(
                        ibuf.at[s, pl.ds(j * 16, 16)],
                        pos_v[s, pl.ds(j * 16, 16)],
                    )
                return carry
            lax.fori_loop(0, SEQ, row_body, 0)

        def chunk_body(c, carry):
            for cp in issue_gathers(c, it0_v, fk0_v, gs0):
                cp.wait()
            pos_add(it0_v)
            for cp in issue_writeback(c, it0_v, fk0_v, ws0):
                cp.wait()
            return carry

        lax.fori_loop(0, B_PER_W, chunk_body, 0)

    return k(ids, fids, item_table, pos_table, fake_table, fake_dummy)


def kernel(input_ids, fake_ids, item_table, pos_table, fake_table):
    ids = input_ids.astype(jnp.int32)
    fids = fake_ids.astype(jnp.int32)
    fake_dummy = jnp.zeros((SEQ, D_FAKE), jnp.float32)
    out = _sc_embed(ids, fids, item_table, pos_table, fake_table, fake_dummy)
    return out.reshape(BATCH, SEQ, D_OUT)
